# baseline (speedup 1.0000x reference)
import jax
import jax.numpy as jnp
from jax import lax
from jax.experimental import pallas as pl
from jax.experimental.pallas import tpu as pltpu

B, T, H, D_BERT = 16, 2048, 512, 1024
NUM_PHONEMES, NUM_TONES, NUM_LANGUAGES = 512, 16, 8
T_BLK = 1024
N_TBLK = T // T_BLK


def _kernel(pid_ref, tlid_ref, feats_ref, ptab_ref, ttab_ref, ltab_ref,
            w_ref, out_ref):
    t_blk = pid_ref.shape[-1]
    w = w_ref[...]
    feats = feats_ref[0].astype(jnp.bfloat16)
    acc = lax.dot_general(w, feats, (((1,), (0,)), ((), ())),
                          preferred_element_type=jnp.float32)

    pid = pid_ref[0, 0, :]
    iota_v = lax.broadcasted_iota(jnp.int32, (NUM_PHONEMES, t_blk), 0)
    onehot_p = (iota_v == pid[None, :]).astype(jnp.bfloat16)
    acc += lax.dot_general(ptab_ref[...], onehot_p, (((1,), (0,)), ((), ())),
                           preferred_element_type=jnp.float32)

    comb = (ttab_ref[...][:, :, None] + ltab_ref[...][:, None, :]).reshape(
        H, NUM_TONES * NUM_LANGUAGES)
    tlid = tlid_ref[0, 0, :]
    iota_tl = lax.broadcasted_iota(
        jnp.int32, (NUM_TONES * NUM_LANGUAGES, t_blk), 0)
    onehot_tl = (iota_tl == tlid[None, :]).astype(jnp.bfloat16)
    acc += lax.dot_general(comb, onehot_tl, (((1,), (0,)), ((), ())),
                           preferred_element_type=jnp.float32)

    out_ref[0] = acc


def kernel(phoneme_ids, tone_ids, language_ids, bert_feats,
           phoneme_table, tone_table, language_table, W_bert):
    ptab_t = phoneme_table.T.astype(jnp.bfloat16)
    ttab_t = tone_table.T.astype(jnp.bfloat16)
    ltab_t = language_table.T.astype(jnp.bfloat16)
    w_bf = W_bert.astype(jnp.bfloat16)
    tl_ids = tone_ids * NUM_LANGUAGES + language_ids

    pid3 = phoneme_ids.reshape(B * N_TBLK, 1, T_BLK)
    tlid3 = tl_ids.reshape(B * N_TBLK, 1, T_BLK)

    grid = (B, N_TBLK)
    out = pl.pallas_call(
        _kernel,
        grid=grid,
        in_specs=[
            pl.BlockSpec((1, 1, T_BLK), lambda b, tb: (b * N_TBLK + tb, 0, 0)),
            pl.BlockSpec((1, 1, T_BLK), lambda b, tb: (b * N_TBLK + tb, 0, 0)),
            pl.BlockSpec((1, D_BERT, T_BLK), lambda b, tb: (b, 0, tb)),
            pl.BlockSpec((H, NUM_PHONEMES), lambda b, tb: (0, 0)),
            pl.BlockSpec((H, NUM_TONES), lambda b, tb: (0, 0)),
            pl.BlockSpec((H, NUM_LANGUAGES), lambda b, tb: (0, 0)),
            pl.BlockSpec((H, D_BERT), lambda b, tb: (0, 0)),
        ],
        out_specs=pl.BlockSpec((1, H, T_BLK), lambda b, tb: (b, 0, tb)),
        out_shape=jax.ShapeDtypeStruct((B, H, T), jnp.float32),
        compiler_params=pltpu.CompilerParams(
            dimension_semantics=("parallel", "parallel"),
        ),
    )(pid3, tlid3, bert_feats, ptab_t, ttab_t, ltab_t, w_bf)
    return out


# speedup vs baseline: 1.1363x; 1.1363x over previous
import jax
import jax.numpy as jnp
from jax import lax
from jax.experimental import pallas as pl
from jax.experimental.pallas import tpu as pltpu

B, T, H, D_BERT = 16, 2048, 512, 1024
NUM_PHONEMES, NUM_TONES, NUM_LANGUAGES = 512, 16, 8
T_BLK = 2048
N_TBLK = T // T_BLK

_CONTRACT = (((1,), (0,)), ((), ()))


def _onehot_dot(tab_ref, ids, vocab, t_blk, acc):
    iota_v = lax.broadcasted_iota(jnp.int32, (vocab, t_blk), 0)
    onehot = (iota_v == ids[None, :]).astype(jnp.bfloat16)
    return acc + lax.dot_general(tab_ref[...], onehot, _CONTRACT,
                                 preferred_element_type=jnp.float32)


def _kernel(pid_ref, tid_ref, lid_ref, feats_ref, ptab_ref, ttab_ref,
            ltab_ref, w_ref, out_ref):
    t_blk = pid_ref.shape[-1]
    feats = feats_ref[0].astype(jnp.bfloat16)
    acc = lax.dot_general(w_ref[...], feats, _CONTRACT,
                          preferred_element_type=jnp.float32)
    acc = _onehot_dot(ptab_ref, pid_ref[0, 0, :], NUM_PHONEMES, t_blk, acc)
    acc = _onehot_dot(ttab_ref, tid_ref[0, 0, :], NUM_TONES, t_blk, acc)
    acc = _onehot_dot(ltab_ref, lid_ref[0, 0, :], NUM_LANGUAGES, t_blk, acc)
    out_ref[0] = acc


def kernel(phoneme_ids, tone_ids, language_ids, bert_feats,
           phoneme_table, tone_table, language_table, W_bert):
    ptab_t = phoneme_table.T.astype(jnp.bfloat16)
    ttab_t = tone_table.T.astype(jnp.bfloat16)
    ltab_t = language_table.T.astype(jnp.bfloat16)
    w_bf = W_bert.astype(jnp.bfloat16)

    pid3 = phoneme_ids.reshape(B * N_TBLK, 1, T_BLK)
    tid3 = tone_ids.reshape(B * N_TBLK, 1, T_BLK)
    lid3 = language_ids.reshape(B * N_TBLK, 1, T_BLK)

    id_spec = pl.BlockSpec((1, 1, T_BLK),
                           lambda b, tb: (b * N_TBLK + tb, 0, 0))
    grid = (B, N_TBLK)
    out = pl.pallas_call(
        _kernel,
        grid=grid,
        in_specs=[
            id_spec,
            id_spec,
            id_spec,
            pl.BlockSpec((1, D_BERT, T_BLK), lambda b, tb: (b, 0, tb)),
            pl.BlockSpec((H, NUM_PHONEMES), lambda b, tb: (0, 0)),
            pl.BlockSpec((H, NUM_TONES), lambda b, tb: (0, 0)),
            pl.BlockSpec((H, NUM_LANGUAGES), lambda b, tb: (0, 0)),
            pl.BlockSpec((H, D_BERT), lambda b, tb: (0, 0)),
        ],
        out_specs=pl.BlockSpec((1, H, T_BLK), lambda b, tb: (b, 0, tb)),
        out_shape=jax.ShapeDtypeStruct((B, H, T), jnp.float32),
        compiler_params=pltpu.CompilerParams(
            dimension_semantics=("parallel", "parallel"),
        ),
    )(pid3, tid3, lid3, bert_feats, ptab_t, ttab_t, ltab_t, w_bf)
    return out


# speedup vs baseline: 1.1793x; 1.0378x over previous
import jax
import jax.numpy as jnp
from jax import lax
from jax.experimental import pallas as pl
from jax.experimental.pallas import tpu as pltpu

B, T, H, D_BERT = 16, 2048, 512, 1024
NUM_PHONEMES, NUM_TONES, NUM_LANGUAGES = 512, 16, 8
T_BLK = 2048
N_TBLK = T // T_BLK

_CONTRACT = (((1,), (0,)), ((), ()))


def _kernel(pid_ref, tlid_ref, feats_ref, ptab_ref, ttab_ref, ltab_ref,
            w_ref, out_ref):
    t_blk = pid_ref.shape[-1]
    feats = feats_ref[0].astype(jnp.bfloat16)
    acc = lax.dot_general(w_ref[...], feats, _CONTRACT,
                          preferred_element_type=jnp.float32)

    pid = pid_ref[0, 0, :]
    iota_v = lax.broadcasted_iota(jnp.int32, (NUM_PHONEMES, t_blk), 0)
    onehot_p = (iota_v == pid[None, :]).astype(jnp.bfloat16)
    acc += lax.dot_general(ptab_ref[...], onehot_p, _CONTRACT,
                           preferred_element_type=jnp.float32)

    comb = (ttab_ref[...][:, :, None] + ltab_ref[...][:, None, :]).reshape(
        H, NUM_TONES * NUM_LANGUAGES)
    tlid = tlid_ref[0, 0, :]
    iota_tl = lax.broadcasted_iota(
        jnp.int32, (NUM_TONES * NUM_LANGUAGES, t_blk), 0)
    onehot_tl = (iota_tl == tlid[None, :]).astype(jnp.bfloat16)
    acc += lax.dot_general(comb, onehot_tl, _CONTRACT,
                           preferred_element_type=jnp.float32)

    out_ref[0] = acc


def kernel(phoneme_ids, tone_ids, language_ids, bert_feats,
           phoneme_table, tone_table, language_table, W_bert):
    ptab_t = phoneme_table.T.astype(jnp.bfloat16)
    ttab_t = tone_table.T.astype(jnp.bfloat16)
    ltab_t = language_table.T.astype(jnp.bfloat16)
    w_bf = W_bert.astype(jnp.bfloat16)
    tl_ids = tone_ids * NUM_LANGUAGES + language_ids

    pid3 = phoneme_ids.reshape(B * N_TBLK, 1, T_BLK)
    tlid3 = tl_ids.reshape(B * N_TBLK, 1, T_BLK)

    id_spec = pl.BlockSpec((1, 1, T_BLK),
                           lambda b, tb: (b * N_TBLK + tb, 0, 0))
    grid = (B, N_TBLK)
    out = pl.pallas_call(
        _kernel,
        grid=grid,
        in_specs=[
            id_spec,
            id_spec,
            pl.BlockSpec((1, D_BERT, T_BLK), lambda b, tb: (b, 0, tb)),
            pl.BlockSpec((H, NUM_PHONEMES), lambda b, tb: (0, 0)),
            pl.BlockSpec((H, NUM_TONES), lambda b, tb: (0, 0)),
            pl.BlockSpec((H, NUM_LANGUAGES), lambda b, tb: (0, 0)),
            pl.BlockSpec((H, D_BERT), lambda b, tb: (0, 0)),
        ],
        out_specs=pl.BlockSpec((1, H, T_BLK), lambda b, tb: (b, 0, tb)),
        out_shape=jax.ShapeDtypeStruct((B, H, T), jnp.float32),
        compiler_params=pltpu.CompilerParams(
            dimension_semantics=("parallel", "parallel"),
        ),
    )(pid3, tlid3, bert_feats, ptab_t, ttab_t, ltab_t, w_bf)
    return out
